# baseline (speedup 1.0000x reference)
import jax
import jax.numpy as jnp
from jax.experimental import pallas as pl
from jax.experimental.pallas import tpu as pltpu

_BLKT = 2048
_BLKK = 256


def _fused_kernel(mask_ref, w_ref, x_ref, b_ref, out_ref, wmix_ref):
    i = pl.program_id(0)
    k = pl.program_id(1)
    e_dim, blkk, _ = w_ref.shape

    @pl.when(i == 0)
    def _mix():
        acc = w_ref[0] * mask_ref[0]
        for e in range(1, e_dim):
            acc += w_ref[e] * mask_ref[e]
        wmix_ref[pl.ds(k * blkk, blkk), :] = acc.astype(jnp.bfloat16)

    xk = x_ref[:, pl.ds(k * blkk, blkk)].astype(jnp.bfloat16)
    partial = jnp.dot(xk, wmix_ref[pl.ds(k * blkk, blkk), :],
                      preferred_element_type=jnp.float32)

    @pl.when(k == 0)
    def _first():
        bmix = b_ref[0:1, :] * mask_ref[0]
        for e in range(1, e_dim):
            bmix += b_ref[e:e + 1, :] * mask_ref[e]
        out_ref[...] = partial + bmix

    @pl.when(k > 0)
    def _rest():
        out_ref[...] += partial


def kernel(x, mask, W, b):
    t, d = x.shape
    e = W.shape[0]
    nt = t // _BLKT
    nk = d // _BLKK

    out = pl.pallas_call(
        _fused_kernel,
        grid=(nt, nk),
        in_specs=[
            pl.BlockSpec(memory_space=pltpu.MemorySpace.SMEM),
            pl.BlockSpec((e, _BLKK, d),
                         lambda i, k: (0, jnp.where(i == 0, k, nk - 1), 0)),
            pl.BlockSpec((_BLKT, d), lambda i, k: (i, 0)),
            pl.BlockSpec((e, d), lambda i, k: (0, 0)),
        ],
        out_specs=pl.BlockSpec((_BLKT, d), lambda i, k: (i, 0)),
        out_shape=jax.ShapeDtypeStruct((t, d), jnp.float32),
        scratch_shapes=[pltpu.VMEM((d, d), jnp.bfloat16)],
    )(mask, W, x, b)

    return (out, mask)


# speedup vs baseline: 1.3322x; 1.3322x over previous
import jax
import jax.numpy as jnp
from jax.experimental import pallas as pl
from jax.experimental.pallas import tpu as pltpu

_BLKW = 256
_BLKT = 1024


def _fused_kernel(mask_ref, w_ref, x_ref, b_ref, out_ref, wmix_ref):
    s = pl.program_id(0)
    e_dim, blkw, _ = w_ref.shape
    nkw = wmix_ref.shape[0] // blkw

    @pl.when(s < nkw)
    def _mix():
        acc = w_ref[0] * mask_ref[0]
        for e in range(1, e_dim):
            acc += w_ref[e] * mask_ref[e]
        wmix_ref[pl.ds(s * blkw, blkw), :] = acc.astype(jnp.bfloat16)

    @pl.when(s >= nkw)
    def _matmul():
        acc = jnp.dot(x_ref[...].astype(jnp.bfloat16), wmix_ref[...],
                      preferred_element_type=jnp.float32)
        bmix = b_ref[0:1, :] * mask_ref[0]
        for e in range(1, e_dim):
            bmix += b_ref[e:e + 1, :] * mask_ref[e]
        out_ref[...] = acc + bmix


def kernel(x, mask, W, b):
    t, d = x.shape
    e = W.shape[0]
    nkw = d // _BLKW
    nt = t // _BLKT

    out = pl.pallas_call(
        _fused_kernel,
        grid=(nkw + nt,),
        in_specs=[
            pl.BlockSpec(memory_space=pltpu.MemorySpace.SMEM),
            pl.BlockSpec((e, _BLKW, d),
                         lambda s: (0, jnp.minimum(s, nkw - 1), 0)),
            pl.BlockSpec((_BLKT, d),
                         lambda s: (jnp.maximum(s - nkw, 0), 0)),
            pl.BlockSpec((e, d), lambda s: (0, 0)),
        ],
        out_specs=pl.BlockSpec((_BLKT, d),
                               lambda s: (jnp.maximum(s - nkw, 0), 0)),
        out_shape=jax.ShapeDtypeStruct((t, d), jnp.float32),
        scratch_shapes=[pltpu.VMEM((d, d), jnp.bfloat16)],
    )(mask, W, x, b)

    return (out, mask)


# speedup vs baseline: 1.3513x; 1.0144x over previous
import jax
import jax.numpy as jnp
from jax.experimental import pallas as pl
from jax.experimental.pallas import tpu as pltpu

_BLKW = 256
_BLKT = 2048


def _fused_kernel(mask_ref, w_ref, x_ref, b_ref, out_ref, wmix_ref):
    s = pl.program_id(0)
    e_dim, blkw, _ = w_ref.shape
    nkw = wmix_ref.shape[0] // blkw

    @pl.when(s < nkw)
    def _mix():
        acc = w_ref[0] * mask_ref[0]
        for e in range(1, e_dim):
            acc += w_ref[e] * mask_ref[e]
        wmix_ref[pl.ds(s * blkw, blkw), :] = acc.astype(jnp.bfloat16)

    @pl.when(s >= nkw)
    def _matmul():
        acc = jnp.dot(x_ref[...].astype(jnp.bfloat16), wmix_ref[...],
                      preferred_element_type=jnp.float32)
        bmix = b_ref[0:1, :] * mask_ref[0]
        for e in range(1, e_dim):
            bmix += b_ref[e:e + 1, :] * mask_ref[e]
        out_ref[...] = acc + bmix


def kernel(x, mask, W, b):
    t, d = x.shape
    e = W.shape[0]
    nkw = d // _BLKW
    nt = t // _BLKT

    out = pl.pallas_call(
        _fused_kernel,
        grid=(nkw + nt,),
        in_specs=[
            pl.BlockSpec(memory_space=pltpu.MemorySpace.SMEM),
            pl.BlockSpec((e, _BLKW, d),
                         lambda s: (0, jnp.minimum(s, nkw - 1), 0)),
            pl.BlockSpec((_BLKT, d),
                         lambda s: (jnp.maximum(s - nkw, 0), 0)),
            pl.BlockSpec((e, d), lambda s: (0, 0)),
        ],
        out_specs=pl.BlockSpec((_BLKT, d),
                               lambda s: (jnp.maximum(s - nkw, 0), 0)),
        out_shape=jax.ShapeDtypeStruct((t, d), jnp.float32),
        scratch_shapes=[pltpu.VMEM((d, d), jnp.bfloat16)],
    )(mask, W, x, b)

    return (out, mask)


# speedup vs baseline: 1.3537x; 1.0018x over previous
import jax
import jax.numpy as jnp
from jax.experimental import pallas as pl
from jax.experimental.pallas import tpu as pltpu

_BLKW = 256
_BLKT = 2048


def _fused_kernel(mask_ref, w_ref, x_ref, b_ref, out_ref, wmix_ref,
                  bmix_ref):
    s = pl.program_id(0)
    e_dim, blkw, _ = w_ref.shape
    nkw = wmix_ref.shape[0] // blkw

    @pl.when(s == 0)
    def _bias():
        bmix = b_ref[0:1, :] * mask_ref[0]
        for e in range(1, e_dim):
            bmix += b_ref[e:e + 1, :] * mask_ref[e]
        bmix_ref[...] = bmix

    @pl.when(s < nkw)
    def _mix():
        acc = w_ref[0] * mask_ref[0]
        for e in range(1, e_dim):
            acc += w_ref[e] * mask_ref[e]
        wmix_ref[pl.ds(s * blkw, blkw), :] = acc.astype(jnp.bfloat16)

    @pl.when(s >= nkw)
    def _matmul():
        acc = jnp.dot(x_ref[...].astype(jnp.bfloat16), wmix_ref[...],
                      preferred_element_type=jnp.float32)
        out_ref[...] = acc + bmix_ref[...]


def kernel(x, mask, W, b):
    t, d = x.shape
    e = W.shape[0]
    nkw = d // _BLKW
    nt = t // _BLKT

    out = pl.pallas_call(
        _fused_kernel,
        grid=(nkw + nt,),
        in_specs=[
            pl.BlockSpec(memory_space=pltpu.MemorySpace.SMEM),
            pl.BlockSpec((e, _BLKW, d),
                         lambda s: (0, jnp.minimum(s, nkw - 1), 0)),
            pl.BlockSpec((_BLKT, d),
                         lambda s: (jnp.maximum(s - nkw, 0), 0)),
            pl.BlockSpec((e, d), lambda s: (0, 0)),
        ],
        out_specs=pl.BlockSpec((_BLKT, d),
                               lambda s: (jnp.maximum(s - nkw, 0), 0)),
        out_shape=jax.ShapeDtypeStruct((t, d), jnp.float32),
        scratch_shapes=[pltpu.VMEM((d, d), jnp.bfloat16),
                        pltpu.VMEM((1, d), jnp.float32)],
    )(mask, W, x, b)

    return (out, mask)
